# baseline (speedup 1.0000x reference)
import functools

import jax
import jax.numpy as jnp
from jax import lax
from jax.experimental import pallas as pl
from jax.experimental.pallas import tpu as pltpu
from jax.experimental.pallas import tpu_sc as plsc

_D = 64
_B = 16384
_N = 1000000
_NC = 2
_NS = 16
_NW = _NC * _NS
_WIN = 128
_NWIN = -(-_N // _WIN)
_WPT = -(-_NWIN // _NW)
_WPAD = 256
_CAP = 16
_ICH = 1024
_NICH = _B // _ICH
_STG = 128


def _gather_call(bidx, tableT):
    mesh = plsc.VectorSubcoreMesh(core_axis_name="c", subcore_axis_name="s")

    @functools.partial(
        pl.kernel,
        mesh=mesh,
        out_type=jax.ShapeDtypeStruct((_B, _D), jnp.float32),
        scratch_types=[
            pltpu.VMEM((_ICH,), jnp.int32),
            pltpu.VMEM((_B + 16,), jnp.int32),
            pltpu.VMEM((_B + 16,), jnp.int32),
            pltpu.VMEM((_B + 16,), jnp.int32),
            pltpu.VMEM((_B + 16,), jnp.int32),
            pltpu.VMEM((_WPAD,), jnp.int32),
            pltpu.VMEM((_WPAD * _CAP,), jnp.int32),
            pltpu.VMEM((_WPAD * _CAP,), jnp.int32),
            pltpu.VMEM((2, _D, _WIN), jnp.float32),
            pltpu.VMEM((_STG, _D), jnp.float32),
            pltpu.SemaphoreType.DMA((2,)),
            pltpu.SemaphoreType.DMA,
        ],
        compiler_params=pltpu.CompilerParams(needs_layout_passes=False),
    )
    def k(bidx_hbm, tableT_hbm, out_hbm, ibuf, lb, lp, ob, op, cnts,
          bkb, bkp, ring, stage, semw, semo):
        wid = lax.axis_index("s") * _NC + lax.axis_index("c")
        lo = wid * _WPT
        iota = lax.iota(jnp.int32, 16)
        zeros16 = jnp.zeros((16,), jnp.int32)

        for t in range(_WPAD // 16):
            cnts[pl.ds(t * 16, 16)] = zeros16

        def filt_chunk(c, ptr):
            pltpu.sync_copy(bidx_hbm.at[pl.ds(c * _ICH, _ICH)], ibuf)

            def filt_vec(v, p):
                b = ibuf[pl.ds(v * 16, 16)]
                w = jnp.right_shift(b, 7)
                wl = w - lo
                m = (wl >= 0) & (wl < _WPT)
                pos = c * _ICH + v * 16 + iota
                csum = plsc.cumsum(m.astype(jnp.int32))
                dest = p + csum - 1
                plsc.store_scatter(lb, [dest], b, mask=m)
                plsc.store_scatter(lp, [dest], pos, mask=m)
                return p + csum[15]

            return lax.fori_loop(0, _ICH // 16, filt_vec, ptr)

        nl = lax.fori_loop(0, _NICH, filt_chunk, jnp.int32(0))

        def bucket_vec(v, optr):
            b = lb[pl.ds(v * 16, 16)]
            pos = lp[pl.ds(v * 16, 16)]
            valid = (v * 16 + iota) < nl
            w = jnp.right_shift(b, 7)
            wl = jnp.where(valid, w - lo, 0)
            cg = plsc.load_gather(cnts, [wl])
            rank = zeros16
            tot = zeros16
            vi = valid.astype(jnp.int32)
            for j in range(16):
                same = ((wl == wl[j]) & valid).astype(jnp.int32) * vi[j]
                rank = rank + same * (iota > j).astype(jnp.int32)
                tot = tot + same
            slot = cg + rank
            main = valid & (slot < _CAP)
            plsc.store_scatter(bkb, [wl * _CAP + slot], b, mask=main)
            plsc.store_scatter(bkp, [wl * _CAP + slot], pos, mask=main)
            plsc.store_scatter(cnts, [wl], cg + tot, mask=valid)
            om = valid & (slot >= _CAP)
            ocs = plsc.cumsum(om.astype(jnp.int32))
            odst = optr + ocs - 1
            plsc.store_scatter(ob, [odst], b, mask=om)
            plsc.store_scatter(op, [odst], pos, mask=om)
            return optr + ocs[15]

        novf = lax.fori_loop(0, _B // 16, bucket_vec, jnp.int32(0))

        def fetch_window(wg, par):
            base = pl.multiple_of(wg * _WIN, _WIN)
            pltpu.async_copy(
                tableT_hbm.at[:, pl.ds(base, _WIN)], ring.at[par],
                semw.at[par])

        def wait_window(par):
            pltpu.make_async_copy(
                tableT_hbm.at[:, pl.ds(0, _WIN)], ring.at[par],
                semw.at[par]).wait()

        def drain_rows(n):
            def d(i, _):
                pltpu.make_async_copy(
                    out_hbm.at[pl.ds(0, 1)], stage.at[pl.ds(0, 1)],
                    semo).wait()
                return 0
            lax.fori_loop(0, n, d, jnp.int32(0))

        def extract_pending(nz, pwl, pcnt, ecnt):
            par = (nz - 1) & 1
            wait_window(par)
            eb = plsc.load_gather(bkb, [pwl * _CAP + iota])
            ep = plsc.load_gather(bkp, [pwl * _CAP + iota])
            lane = eb & 127
            em = iota < pcnt
            pvec = jnp.broadcast_to(par, (16,))
            rows = ecnt + iota
            for c in range(_D):
                vals = plsc.load_gather(
                    ring, [pvec, jnp.broadcast_to(c, (16,)), lane])
                plsc.store_scatter(stage, [rows, jnp.broadcast_to(c, (16,))],
                                   vals, mask=em)
            for j in range(_CAP):
                @pl.when(j < pcnt)
                def _():
                    pltpu.async_copy(
                        stage.at[ecnt + j], out_hbm.at[ep[j]], semo)

        def walk(t, carry):
            nz, pwl, pcnt, ecnt, fcnt = carry
            cvec = cnts[pl.ds(t * 16, 16)]
            for j in range(16):
                wl = t * 16 + j
                cnt = jnp.minimum(cvec[j], _CAP)
                go = cnt > 0
                need_flush = go & (ecnt + _CAP > _STG)

                @pl.when(need_flush)
                def _():
                    drain_rows(fcnt)
                ecnt = jnp.where(need_flush, 0, ecnt)
                fcnt = jnp.where(need_flush, 0, fcnt)

                @pl.when(go)
                def _():
                    fetch_window(lo + wl, nz & 1)

                has_pending = go & (nz > 0)

                @pl.when(has_pending)
                def _():
                    extract_pending(nz, pwl, pcnt, ecnt)
                ecnt = jnp.where(has_pending, ecnt + pcnt, ecnt)
                fcnt = jnp.where(has_pending, fcnt + pcnt, fcnt)
                nz = jnp.where(go, nz + 1, nz)
                pwl = jnp.where(go, jnp.int32(wl), pwl)
                pcnt = jnp.where(go, cnt, pcnt)
            return nz, pwl, pcnt, ecnt, fcnt

        z = jnp.int32(0)
        nz, pwl, pcnt, ecnt, fcnt = lax.fori_loop(
            0, _WPAD // 16, walk, (z, z, z, z, z))

        last_flush = (nz > 0) & (ecnt + _CAP > _STG)

        @pl.when(last_flush)
        def _():
            drain_rows(fcnt)
        ecnt = jnp.where(last_flush, 0, ecnt)
        fcnt = jnp.where(last_flush, 0, fcnt)

        @pl.when(nz > 0)
        def _():
            extract_pending(nz, pwl, pcnt, ecnt)
        fcnt = jnp.where(nz > 0, fcnt + pcnt, fcnt)

        @pl.when(fcnt > 0)
        def _():
            drain_rows(fcnt)

        @pl.when(novf > 0)
        def _():
            def ovf_chunk(v, _):
                b16 = ob[pl.ds(v * 16, 16)]
                p16 = op[pl.ds(v * 16, 16)]
                for j in range(16):
                    @pl.when((v * 16 + j) < novf)
                    def _():
                        b = b16[j]
                        wg = jnp.right_shift(b, 7)
                        base = pl.multiple_of(wg * _WIN, _WIN)
                        pltpu.async_copy(
                            tableT_hbm.at[:, pl.ds(base, _WIN)],
                            ring.at[0], semw.at[0])
                        wait_window(0)
                        lane = jnp.broadcast_to(b & 127, (16,))
                        for c4 in range(_D // 16):
                            vals = plsc.load_gather(
                                ring, [zeros16, c4 * 16 + iota, lane])
                            stage[0, pl.ds(c4 * 16, 16)] = vals
                        pltpu.async_copy(
                            stage.at[0], out_hbm.at[p16[j]], semo)
                        pltpu.make_async_copy(
                            out_hbm.at[pl.ds(0, 1)],
                            stage.at[pl.ds(0, 1)], semo).wait()
                return 0

            lax.fori_loop(0, -(-novf // 16), ovf_chunk, jnp.int32(0))

    return k(bidx, tableT)


def kernel(batch, weight):
    bidx = batch.astype(jnp.int32)
    return _gather_call(bidx, weight.T)


# speedup vs baseline: 1.2440x; 1.2440x over previous
import functools

import jax
import jax.numpy as jnp
from jax import lax
from jax.experimental import pallas as pl
from jax.experimental.pallas import tpu as pltpu
from jax.experimental.pallas import tpu_sc as plsc

_D = 64
_B = 16384
_N = 1000000
_NC = 2
_NS = 16
_NW = _NC * _NS
_WIN = 128
_NWIN = -(-_N // _WIN)
_WPT = -(-_NWIN // _NW)
_WPAD = 256
_CAP = 16
_ICH = 1024
_NICH = _B // _ICH
_STG = 128


def _gather_call(bidx, tableT):
    mesh = plsc.VectorSubcoreMesh(core_axis_name="c", subcore_axis_name="s")

    @functools.partial(
        pl.kernel,
        mesh=mesh,
        out_type=jax.ShapeDtypeStruct((_B, _D), jnp.float32),
        scratch_types=[
            pltpu.VMEM((_ICH,), jnp.int32),
            pltpu.VMEM((_B + 16,), jnp.int32),
            pltpu.VMEM((_B + 16,), jnp.int32),
            pltpu.VMEM((_B + 16,), jnp.int32),
            pltpu.VMEM((_B + 16,), jnp.int32),
            pltpu.VMEM((_WPAD + 16,), jnp.int32),
            pltpu.VMEM((_WPAD * _CAP,), jnp.int32),
            pltpu.VMEM((_WPAD * _CAP,), jnp.int32),
            pltpu.VMEM((4, _D, _WIN), jnp.float32),
            pltpu.VMEM((_STG, _D), jnp.float32),
            pltpu.SemaphoreType.DMA((4,)),
            pltpu.SemaphoreType.DMA,
        ],
        compiler_params=pltpu.CompilerParams(needs_layout_passes=False),
    )
    def k(bidx_hbm, tableT_hbm, out_hbm, ibuf, lb, lp, ob, op, cnts,
          bkb, bkp, ring, stage, semw, semo):
        wid = lax.axis_index("s") * _NC + lax.axis_index("c")
        lo = wid * _WPT
        iota = lax.iota(jnp.int32, 16)
        zeros16 = jnp.zeros((16,), jnp.int32)

        for t in range((_WPAD + 16) // 16):
            cnts[pl.ds(t * 16, 16)] = zeros16

        def filt_chunk(c, ptr):
            pltpu.sync_copy(bidx_hbm.at[pl.ds(c * _ICH, _ICH)], ibuf)

            def filt_vec(v, p):
                b = ibuf[pl.ds(v * 16, 16)]
                w = jnp.right_shift(b, 7)
                wl = w - lo
                m = (wl >= 0) & (wl < _WPT)
                pos = c * _ICH + v * 16 + iota
                csum = plsc.cumsum(m.astype(jnp.int32))
                dest = p + csum - 1
                plsc.store_scatter(lb, [dest], b, mask=m)
                plsc.store_scatter(lp, [dest], pos, mask=m)
                return p + csum[15]

            return lax.fori_loop(0, _ICH // 16, filt_vec, ptr)

        nl = lax.fori_loop(0, _NICH, filt_chunk, jnp.int32(0))

        def bucket_vec(v, optr):
            b = lb[pl.ds(v * 16, 16)]
            pos = lp[pl.ds(v * 16, 16)]
            valid = (v * 16 + iota) < nl
            w = jnp.right_shift(b, 7)
            wl = jnp.where(valid, w - lo, 0)
            cg = plsc.load_gather(cnts, [wl])
            rank = zeros16
            tot = zeros16
            vi = valid.astype(jnp.int32)
            for j in range(16):
                same = ((wl == wl[j]) & valid).astype(jnp.int32) * vi[j]
                rank = rank + same * (iota > j).astype(jnp.int32)
                tot = tot + same
            slot = cg + rank
            main = valid & (slot < _CAP)
            plsc.store_scatter(bkb, [wl * _CAP + slot], b, mask=main)
            plsc.store_scatter(bkp, [wl * _CAP + slot], pos, mask=main)
            plsc.store_scatter(cnts, [wl], cg + tot, mask=valid)
            om = valid & (slot >= _CAP)
            ocs = plsc.cumsum(om.astype(jnp.int32))
            odst = optr + ocs - 1
            plsc.store_scatter(ob, [odst], b, mask=om)
            plsc.store_scatter(op, [odst], pos, mask=om)
            return optr + ocs[15]

        nvreg = jnp.right_shift(nl + 15, 4)
        novf = lax.fori_loop(0, nvreg, bucket_vec, jnp.int32(0))

        def fetch_window(wg, par):
            base = pl.multiple_of(wg * _WIN, _WIN)
            pltpu.async_copy(
                tableT_hbm.at[:, pl.ds(base, _WIN)], ring.at[par],
                semw.at[par])

        def wait_window(par):
            pltpu.make_async_copy(
                tableT_hbm.at[:, pl.ds(0, _WIN)], ring.at[par],
                semw.at[par]).wait()

        def drain_rows(n):
            def d(i, _):
                pltpu.make_async_copy(
                    out_hbm.at[pl.ds(0, 1)], stage.at[pl.ds(0, 1)],
                    semo).wait()
                return 0
            lax.fori_loop(0, n, d, jnp.int32(0))

        def extract_pending(par, pwl, pcnt, ecnt):
            wait_window(par)
            eb = plsc.load_gather(bkb, [pwl * _CAP + iota])
            ep = plsc.load_gather(bkp, [pwl * _CAP + iota])
            pvec = jnp.broadcast_to(par, (16,))
            for j in range(_CAP):
                @pl.when(j < pcnt)
                def _():
                    lane = jnp.broadcast_to(eb[j] & 127, (16,))
                    row = ecnt + j
                    for c4 in range(_D // 16):
                        vals = plsc.load_gather(
                            ring, [pvec, c4 * 16 + iota, lane])
                        stage[row, pl.ds(c4 * 16, 16)] = vals
                    pltpu.async_copy(
                        stage.at[row], out_hbm.at[ep[j]], semo)

        def walk(t, carry):
            nz, p1wl, p1cnt, p2wl, p2cnt, ecnt, fcnt = carry
            cvec = cnts[pl.ds(t * 8, 16)]
            for j in range(8):
                wl = t * 8 + j
                cnt = jnp.minimum(cvec[j], _CAP)
                go = cnt > 0
                need_flush = go & (ecnt + _CAP > _STG)

                @pl.when(need_flush)
                def _():
                    drain_rows(fcnt)
                ecnt = jnp.where(need_flush, 0, ecnt)
                fcnt = jnp.where(need_flush, 0, fcnt)

                @pl.when(go)
                def _():
                    fetch_window(lo + wl, nz & 3)

                ready = go & (nz >= 2)

                @pl.when(ready)
                def _():
                    extract_pending((nz - 2) & 3, p2wl, p2cnt, ecnt)
                ecnt = jnp.where(ready, ecnt + p2cnt, ecnt)
                fcnt = jnp.where(ready, fcnt + p2cnt, fcnt)
                p2wl = jnp.where(go, p1wl, p2wl)
                p2cnt = jnp.where(go, p1cnt, p2cnt)
                p1wl = jnp.where(go, jnp.int32(wl), p1wl)
                p1cnt = jnp.where(go, cnt, p1cnt)
                nz = jnp.where(go, nz + 1, nz)
            return nz, p1wl, p1cnt, p2wl, p2cnt, ecnt, fcnt

        z = jnp.int32(0)
        nz, p1wl, p1cnt, p2wl, p2cnt, ecnt, fcnt = lax.fori_loop(
            0, _WPAD // 8, walk, (z, z, z, z, z, z, z))

        for which in (2, 1):
            havep = nz >= which
            pwl = p2wl if which == 2 else p1wl
            pcnt = p2cnt if which == 2 else p1cnt
            tail_flush = havep & (ecnt + _CAP > _STG)

            @pl.when(tail_flush)
            def _():
                drain_rows(fcnt)
            ecnt = jnp.where(tail_flush, 0, ecnt)
            fcnt = jnp.where(tail_flush, 0, fcnt)

            @pl.when(havep)
            def _():
                extract_pending((nz - which) & 3, pwl, pcnt, ecnt)
            ecnt = jnp.where(havep, ecnt + pcnt, ecnt)
            fcnt = jnp.where(havep, fcnt + pcnt, fcnt)

        @pl.when(fcnt > 0)
        def _():
            drain_rows(fcnt)

        @pl.when(novf > 0)
        def _():
            def ovf_chunk(v, _):
                b16 = ob[pl.ds(v * 16, 16)]
                p16 = op[pl.ds(v * 16, 16)]
                for j in range(16):
                    @pl.when((v * 16 + j) < novf)
                    def _():
                        b = b16[j]
                        wg = jnp.right_shift(b, 7)
                        base = pl.multiple_of(wg * _WIN, _WIN)
                        pltpu.async_copy(
                            tableT_hbm.at[:, pl.ds(base, _WIN)],
                            ring.at[0], semw.at[0])
                        wait_window(0)
                        lane = jnp.broadcast_to(b & 127, (16,))
                        for c4 in range(_D // 16):
                            vals = plsc.load_gather(
                                ring, [zeros16, c4 * 16 + iota, lane])
                            stage[0, pl.ds(c4 * 16, 16)] = vals
                        pltpu.async_copy(
                            stage.at[0], out_hbm.at[p16[j]], semo)
                        pltpu.make_async_copy(
                            out_hbm.at[pl.ds(0, 1)],
                            stage.at[pl.ds(0, 1)], semo).wait()
                return 0

            lax.fori_loop(0, -(-novf // 16), ovf_chunk, jnp.int32(0))

    return k(bidx, tableT)


def kernel(batch, weight):
    bidx = batch.astype(jnp.int32)
    return _gather_call(bidx, weight.T)


# speedup vs baseline: 1.2986x; 1.0439x over previous
import functools

import jax
import jax.numpy as jnp
from jax import lax
from jax.experimental import pallas as pl
from jax.experimental.pallas import tpu as pltpu
from jax.experimental.pallas import tpu_sc as plsc

_D = 64
_B = 16384
_N = 1000000
_NC = 2
_NS = 16
_NW = _NC * _NS
_WIN = 128
_NWIN = -(-_N // _WIN)
_WPT = -(-_NWIN // _NW)
_WPAD = 256
_CAP = 16
_ICH = 1024
_NICH = _B // _ICH
_STG = 128


def _gather_call(bidx, tableT):
    mesh = plsc.VectorSubcoreMesh(core_axis_name="c", subcore_axis_name="s")

    @functools.partial(
        pl.kernel,
        mesh=mesh,
        out_type=jax.ShapeDtypeStruct((_B, _D), jnp.float32),
        scratch_types=[
            pltpu.VMEM((2, _ICH), jnp.int32),
            pltpu.VMEM((_B + 16,), jnp.int32),
            pltpu.VMEM((_B + 16,), jnp.int32),
            pltpu.VMEM((_B + 16,), jnp.int32),
            pltpu.VMEM((_B + 16,), jnp.int32),
            pltpu.VMEM((_WPAD + 16,), jnp.int32),
            pltpu.VMEM((_WPAD * _CAP,), jnp.int32),
            pltpu.VMEM((_WPAD * _CAP,), jnp.int32),
            pltpu.VMEM((4, _D, _WIN), jnp.float32),
            pltpu.VMEM((_STG, _D), jnp.float32),
            pltpu.SemaphoreType.DMA((4,)),
            pltpu.SemaphoreType.DMA,
        ],
        compiler_params=pltpu.CompilerParams(needs_layout_passes=False),
    )
    def k(bidx_hbm, tableT_hbm, out_hbm, ibuf, lb, lp, ob, op, cnts,
          bkb, bkp, ring, stage, semw, semo):
        wid = lax.axis_index("s") * _NC + lax.axis_index("c")
        lo = wid * _WPT
        iota = lax.iota(jnp.int32, 16)
        zeros16 = jnp.zeros((16,), jnp.int32)

        for t in range((_WPAD + 16) // 16):
            cnts[pl.ds(t * 16, 16)] = zeros16

        def fetch_idx(c, par):
            pltpu.async_copy(
                bidx_hbm.at[pl.ds(c * _ICH, _ICH)], ibuf.at[par],
                semw.at[par])

        def wait_idx(par):
            pltpu.make_async_copy(
                bidx_hbm.at[pl.ds(0, _ICH)], ibuf.at[par],
                semw.at[par]).wait()

        fetch_idx(0, 0)

        def filt_chunk(c, ptr):
            par = c & 1
            wait_idx(par)

            @pl.when(c + 1 < _NICH)
            def _():
                fetch_idx(c + 1, (c + 1) & 1)

            def filt_vec(v, p):
                b = ibuf[par, pl.ds(v * 16, 16)]
                w = jnp.right_shift(b, 7)
                wl = w - lo
                m = (wl >= 0) & (wl < _WPT)
                pos = c * _ICH + v * 16 + iota
                csum = plsc.cumsum(m.astype(jnp.int32))
                dest = p + csum - 1
                plsc.store_scatter(lb, [dest], b, mask=m)
                plsc.store_scatter(lp, [dest], pos, mask=m)
                return p + csum[15]

            return plsc.parallel_loop(
                0, _ICH // 16, unroll=4, carry=ptr)(filt_vec)

        nl = lax.fori_loop(0, _NICH, filt_chunk, jnp.int32(0))

        def bucket_vec(v, optr):
            b = lb[pl.ds(v * 16, 16)]
            pos = lp[pl.ds(v * 16, 16)]
            valid = (v * 16 + iota) < nl
            w = jnp.right_shift(b, 7)
            wl = jnp.where(valid, w - lo, 0)
            cg = plsc.load_gather(cnts, [wl])
            rank = zeros16
            tot = zeros16
            vi = valid.astype(jnp.int32)
            for j in range(16):
                same = ((wl == wl[j]) & valid).astype(jnp.int32) * vi[j]
                rank = rank + same * (iota > j).astype(jnp.int32)
                tot = tot + same
            slot = cg + rank
            main = valid & (slot < _CAP)
            plsc.store_scatter(bkb, [wl * _CAP + slot], b, mask=main)
            plsc.store_scatter(bkp, [wl * _CAP + slot], pos, mask=main)
            plsc.store_scatter(cnts, [wl], cg + tot, mask=valid)
            om = valid & (slot >= _CAP)
            ocs = plsc.cumsum(om.astype(jnp.int32))
            odst = optr + ocs - 1
            plsc.store_scatter(ob, [odst], b, mask=om)
            plsc.store_scatter(op, [odst], pos, mask=om)
            return optr + ocs[15]

        nvreg = jnp.right_shift(nl + 15, 4)
        novf = lax.fori_loop(0, nvreg, bucket_vec, jnp.int32(0))

        def fetch_window(wg, par):
            base = pl.multiple_of(wg * _WIN, _WIN)
            pltpu.async_copy(
                tableT_hbm.at[:, pl.ds(base, _WIN)], ring.at[par],
                semw.at[par])

        def wait_window(par):
            pltpu.make_async_copy(
                tableT_hbm.at[:, pl.ds(0, _WIN)], ring.at[par],
                semw.at[par]).wait()

        def drain_rows(n):
            def d(i, _):
                pltpu.make_async_copy(
                    out_hbm.at[pl.ds(0, 1)], stage.at[pl.ds(0, 1)],
                    semo).wait()
                return 0
            lax.fori_loop(0, n, d, jnp.int32(0))

        def extract_pending(par, pwl, pcnt, ecnt):
            wait_window(par)
            eb = plsc.load_gather(bkb, [pwl * _CAP + iota])
            ep = plsc.load_gather(bkp, [pwl * _CAP + iota])
            pvec = jnp.broadcast_to(par, (16,))
            for j in range(_CAP):
                @pl.when(j < pcnt)
                def _():
                    lane = jnp.broadcast_to(eb[j] & 127, (16,))
                    row = ecnt + j
                    for c4 in range(_D // 16):
                        vals = plsc.load_gather(
                            ring, [pvec, c4 * 16 + iota, lane])
                        stage[row, pl.ds(c4 * 16, 16)] = vals
                    pltpu.async_copy(
                        stage.at[row], out_hbm.at[ep[j]], semo)

        def walk(t, carry):
            nz, p1wl, p1cnt, p2wl, p2cnt, ecnt, fcnt = carry
            cvec = cnts[pl.ds(t * 8, 16)]
            for j in range(8):
                wl = t * 8 + j
                cnt = jnp.minimum(cvec[j], _CAP)
                go = cnt > 0
                need_flush = go & (ecnt + _CAP > _STG)

                @pl.when(need_flush)
                def _():
                    drain_rows(fcnt)
                ecnt = jnp.where(need_flush, 0, ecnt)
                fcnt = jnp.where(need_flush, 0, fcnt)

                @pl.when(go)
                def _():
                    fetch_window(lo + wl, nz & 3)

                ready = go & (nz >= 2)

                @pl.when(ready)
                def _():
                    extract_pending((nz - 2) & 3, p2wl, p2cnt, ecnt)
                ecnt = jnp.where(ready, ecnt + p2cnt, ecnt)
                fcnt = jnp.where(ready, fcnt + p2cnt, fcnt)
                p2wl = jnp.where(go, p1wl, p2wl)
                p2cnt = jnp.where(go, p1cnt, p2cnt)
                p1wl = jnp.where(go, jnp.int32(wl), p1wl)
                p1cnt = jnp.where(go, cnt, p1cnt)
                nz = jnp.where(go, nz + 1, nz)
            return nz, p1wl, p1cnt, p2wl, p2cnt, ecnt, fcnt

        z = jnp.int32(0)
        nz, p1wl, p1cnt, p2wl, p2cnt, ecnt, fcnt = lax.fori_loop(
            0, _WPAD // 8, walk, (z, z, z, z, z, z, z))

        for which in (2, 1):
            havep = nz >= which
            pwl = p2wl if which == 2 else p1wl
            pcnt = p2cnt if which == 2 else p1cnt
            tail_flush = havep & (ecnt + _CAP > _STG)

            @pl.when(tail_flush)
            def _():
                drain_rows(fcnt)
            ecnt = jnp.where(tail_flush, 0, ecnt)
            fcnt = jnp.where(tail_flush, 0, fcnt)

            @pl.when(havep)
            def _():
                extract_pending((nz - which) & 3, pwl, pcnt, ecnt)
            ecnt = jnp.where(havep, ecnt + pcnt, ecnt)
            fcnt = jnp.where(havep, fcnt + pcnt, fcnt)

        @pl.when(fcnt > 0)
        def _():
            drain_rows(fcnt)

        @pl.when(novf > 0)
        def _():
            def ovf_chunk(v, _):
                b16 = ob[pl.ds(v * 16, 16)]
                p16 = op[pl.ds(v * 16, 16)]
                for j in range(16):
                    @pl.when((v * 16 + j) < novf)
                    def _():
                        b = b16[j]
                        wg = jnp.right_shift(b, 7)
                        base = pl.multiple_of(wg * _WIN, _WIN)
                        pltpu.async_copy(
                            tableT_hbm.at[:, pl.ds(base, _WIN)],
                            ring.at[0], semw.at[0])
                        wait_window(0)
                        lane = jnp.broadcast_to(b & 127, (16,))
                        for c4 in range(_D // 16):
                            vals = plsc.load_gather(
                                ring, [zeros16, c4 * 16 + iota, lane])
                            stage[0, pl.ds(c4 * 16, 16)] = vals
                        pltpu.async_copy(
                            stage.at[0], out_hbm.at[p16[j]], semo)
                        pltpu.make_async_copy(
                            out_hbm.at[pl.ds(0, 1)],
                            stage.at[pl.ds(0, 1)], semo).wait()
                return 0

            lax.fori_loop(0, -(-novf // 16), ovf_chunk, jnp.int32(0))

    return k(bidx, tableT)


def kernel(batch, weight):
    bidx = batch.astype(jnp.int32)
    return _gather_call(bidx, weight.T)
